# Initial kernel scaffold; baseline (speedup 1.0000x reference)
import functools
import math

import jax
import jax.numpy as jnp
from jax import lax
from jax.experimental import pallas as pl
from jax.experimental.pallas import tpu as pltpu
from jax.experimental.pallas import tpu_sc as plsc

_D = 32
_SCALE = math.sqrt(_D)

_INFO = plsc.get_sparse_core_info()
_NC = _INFO.num_cores
_NS = _INFO.num_subcores
_NW = _NC * _NS

_CHUNK = 1024
_GSUB = 128
_NSUB = _CHUNK // _GSUB


def _make_kernel(n_rows: int):
  assert n_rows % (_NW * _CHUNK) == 0
  rows_per_w = n_rows // _NW
  n_chunks = rows_per_w // _CHUNK
  mesh = plsc.VectorSubcoreMesh(core_axis_name="c", subcore_axis_name="s")

  @functools.partial(
      pl.kernel,
      mesh=mesh,
      out_type=jax.ShapeDtypeStruct((n_rows, _D), jnp.float32),
      scratch_types=[
          pltpu.VMEM((_CHUNK,), jnp.int32),
          pltpu.VMEM((_CHUNK, _D), jnp.float32),
          pltpu.SemaphoreType.DMA,
      ],
  )
  def body(table_hbm, idx_hbm, out_hbm, idx_v, rows_v, sem):
    wid = lax.axis_index("s") * _NC + lax.axis_index("c")
    wbase = wid * rows_per_w

    def chunk_step(g, carry):
      base = wbase + g * _CHUNK
      pltpu.sync_copy(idx_hbm.at[pl.ds(base, _CHUNK)], idx_v)
      for j in range(_NSUB):
        pltpu.async_copy(
            table_hbm.at[idx_v.at[pl.ds(j * _GSUB, _GSUB)]],
            rows_v.at[pl.ds(j * _GSUB, _GSUB), :],
            sem,
        )
      for j in range(_NSUB):
        pltpu.make_async_copy(
            table_hbm.at[idx_v.at[pl.ds(j * _GSUB, _GSUB)]],
            rows_v.at[pl.ds(j * _GSUB, _GSUB), :],
            sem,
        ).wait()

      def scale_row(i, c):
        for h in range(_D // 16):
          sl = pl.ds(h * 16, 16)
          rows_v[i, sl] = rows_v[i, sl] * _SCALE
        return c

      lax.fori_loop(0, _CHUNK, scale_row, 0, unroll=4)
      pltpu.sync_copy(rows_v, out_hbm.at[pl.ds(base, _CHUNK), :])
      return carry

    lax.fori_loop(0, n_chunks, chunk_step, 0)

  return body


@jax.jit
def kernel(x, embedding):
  b, s = x.shape
  flat_idx = x.reshape(b * s)
  out = _make_kernel(b * s)(embedding, flat_idx)
  return out.reshape(b, s, _D)


# speedup vs baseline: 4.5691x; 4.5691x over previous
import functools
import math

import jax
import jax.numpy as jnp
from jax import lax
from jax.experimental import pallas as pl
from jax.experimental.pallas import tpu as pltpu
from jax.experimental.pallas import tpu_sc as plsc

_D = 32
_SCALE = math.sqrt(_D)

_INFO = plsc.get_sparse_core_info()
_NC = _INFO.num_cores
_NS = _INFO.num_subcores
_NW = _NC * _NS

_CHUNK = 1024
_GSUB = 128
_NSUB = _CHUNK // _GSUB


def _make_kernel(n_rows: int):
  assert n_rows % (_NW * _CHUNK) == 0
  rows_per_w = n_rows // _NW
  n_chunks = rows_per_w // _CHUNK
  mesh = plsc.VectorSubcoreMesh(core_axis_name="c", subcore_axis_name="s")

  @functools.partial(
      pl.kernel,
      mesh=mesh,
      out_type=jax.ShapeDtypeStruct((n_rows, _D), jnp.float32),
      scratch_types=[
          pltpu.VMEM((_CHUNK,), jnp.int32),
          pltpu.VMEM((_CHUNK, _D), jnp.float32),
          pltpu.SemaphoreType.DMA,
      ],
      compiler_params=pltpu.CompilerParams(use_tc_tiling_on_sc=False),
  )
  def body(table_hbm, idx_hbm, out_hbm, idx_v, rows_v, sem):
    wid = lax.axis_index("s") * _NC + lax.axis_index("c")
    wbase = wid * rows_per_w

    def chunk_step(g, carry):
      base = wbase + g * _CHUNK
      pltpu.sync_copy(idx_hbm.at[pl.ds(base, _CHUNK)], idx_v)
      for j in range(_NSUB):
        pltpu.async_copy(
            table_hbm.at[idx_v.at[pl.ds(j * _GSUB, _GSUB)]],
            rows_v.at[pl.ds(j * _GSUB, _GSUB), :],
            sem,
        )
      for j in range(_NSUB):
        pltpu.make_async_copy(
            table_hbm.at[idx_v.at[pl.ds(j * _GSUB, _GSUB)]],
            rows_v.at[pl.ds(j * _GSUB, _GSUB), :],
            sem,
        ).wait()

      def scale_row(i, c):
        for h in range(_D // 16):
          sl = pl.ds(h * 16, 16)
          rows_v[i, sl] = rows_v[i, sl] * _SCALE
        return c

      lax.fori_loop(0, _CHUNK, scale_row, 0, unroll=4)
      pltpu.sync_copy(rows_v, out_hbm.at[pl.ds(base, _CHUNK), :])
      return carry

    lax.fori_loop(0, n_chunks, chunk_step, 0)

  return body


@jax.jit
def kernel(x, embedding):
  b, s = x.shape
  flat_idx = x.reshape(b * s)
  out = _make_kernel(b * s)(embedding, flat_idx)
  return out.reshape(b, s, _D)
